# baseline (speedup 1.0000x reference)
import functools

import jax
import jax.numpy as jnp
from jax import lax
from jax.experimental import pallas as pl
from jax.experimental.pallas import tpu as pltpu
from jax.experimental.pallas import tpu_sc as plsc

W = 1280
H = 720
WH = W * H
TWO = 2 * WH
N = 2_000_000
C = 640
R = C // 128
NCHUNK = N // C
NW = 32
SLICE = TWO // 16
ZC = 7200


def _scatter_body(ex, ey, ep, zeros_in, ones_in, masks, shared,
                  xa, ya, pa, xbb, yb, pb, ia, ib, ones_v,
                  lsa, lsb, ssa, ssb):
    c = lax.axis_index("c")
    s = lax.axis_index("s")
    wid = s * 2 + c

    start = wid * NCHUNK // NW
    end = (wid + 1) * NCHUNK // NW
    cnt = end - start

    def fire_loads(j, xr, yr, pr, ls):
        base = j * C
        pltpu.async_copy(ex.at[pl.ds(base, C)], xr, ls)
        pltpu.async_copy(ey.at[pl.ds(base, C)], yr, ls)
        pltpu.async_copy(ep.at[pl.ds(base, C)], pr, ls)

    fire_loads(start, xa, ya, pa, lsa)
    fire_loads(start + 1, xbb, yb, pb, lsb)

    pltpu.sync_copy(ones_in, ones_v)

    pltpu.sync_copy(zeros_in, shared.at[pl.ds(s * SLICE, SLICE)])
    plsc.subcore_barrier()

    def step(j, xr, yr, pr, idxr, ls, ss, first):
        pltpu.make_async_copy(ex.at[pl.ds(0, C)], xr, ls).wait()
        pltpu.make_async_copy(ex.at[pl.ds(0, C)], yr, ls).wait()
        pltpu.make_async_copy(ex.at[pl.ds(0, C)], pr, ls).wait()

        @pl.when(jnp.logical_not(first))
        def _():
            pltpu.make_async_copy(ex.at[pl.ds(0, C)], xr, ss).wait()

        for r in range(R):
            for i8 in range(8):
                off = r * 128 + i8 * 16
                xv = xr[pl.ds(off, 16)]
                yv = yr[pl.ds(off, 16)]
                pv = pr[pl.ds(off, 16)]
                idx = pv * WH + xv * H + yv
                idxr[r, pl.ds(i8 * 16, 16)] = idx
        for r in range(R):
            pltpu.async_copy(ones_v, shared.at[idxr.at[r]], ss)

        @pl.when(j + 2 < end)
        def _():
            fire_loads(j + 2, xr, yr, pr, ls)

    def pair_body(t2, carry):
        j = start + 2 * t2
        step(j, xa, ya, pa, ia, lsa, ssa, t2 == 0)
        step(j + 1, xbb, yb, pb, ib, lsb, ssb, t2 == 0)
        return carry

    lax.fori_loop(0, cnt // 2, pair_body, 0)

    @pl.when(cnt % 2 == 1)
    def _():
        jt = start + 2 * (cnt // 2)
        step(jt, xa, ya, pa, ia, lsa, ssa, jnp.bool_(False))

    pltpu.make_async_copy(ex.at[pl.ds(0, C)], xa, ssa).wait()
    pltpu.make_async_copy(ex.at[pl.ds(0, C)], xbb, ssb).wait()
    plsc.subcore_barrier()

    pltpu.sync_copy(shared.at[pl.ds(s * SLICE, SLICE)],
                    masks.at[pl.ds(c * TWO + s * SLICE, SLICE)])


@functools.partial(
    pl.kernel,
    out_type=jax.ShapeDtypeStruct((2 * TWO,), jnp.int32),
    mesh=plsc.VectorSubcoreMesh(core_axis_name="c", subcore_axis_name="s"),
    scratch_types=[
        pltpu.VMEM_SHARED((TWO,), jnp.int32),
        pltpu.VMEM((C,), jnp.int32),
        pltpu.VMEM((C,), jnp.int32),
        pltpu.VMEM((C,), jnp.int32),
        pltpu.VMEM((C,), jnp.int32),
        pltpu.VMEM((C,), jnp.int32),
        pltpu.VMEM((C,), jnp.int32),
        pltpu.VMEM((R, 128), jnp.int32),
        pltpu.VMEM((R, 128), jnp.int32),
        pltpu.VMEM((128,), jnp.int32),
        pltpu.SemaphoreType.DMA,
        pltpu.SemaphoreType.DMA,
        pltpu.SemaphoreType.DMA,
        pltpu.SemaphoreType.DMA,
    ],
)
def _scatter_masks(ex, ey, ep, zeros_in, ones_in, masks, shared,
                   xa, ya, pa, xbb, yb, pb, ia, ib, ones_v,
                   lsa, lsb, ssa, ssb):
    _scatter_body(ex, ey, ep, zeros_in, ones_in, masks, shared,
                  xa, ya, pa, xbb, yb, pb, ia, ib, ones_v,
                  lsa, lsb, ssa, ssb)


def _combine_body(m_ref, pic_ref, o_ref):
    t0 = jnp.maximum(m_ref[0, 0], m_ref[1, 0]).astype(jnp.float32)
    t1 = jnp.maximum(m_ref[0, 1], m_ref[1, 1]).astype(jnp.float32)
    o_ref[...] = pic_ref[...] + 15.0 * t1 - 15.0 * t0


def kernel(events_x, events_y, events_polarity, pic_tensor):
    zeros_in = jnp.zeros((SLICE,), jnp.int32)
    ones_in = jnp.ones((128,), jnp.int32)
    masks = _scatter_masks(events_x, events_y, events_polarity, zeros_in,
                           ones_in)
    m = masks.reshape(2, 2, W, H)
    BW = 128
    out = pl.pallas_call(
        _combine_body,
        grid=(W // BW,),
        in_specs=[
            pl.BlockSpec((2, 2, BW, H), lambda i: (0, 0, i, 0)),
            pl.BlockSpec((BW, H), lambda i: (i, 0)),
        ],
        out_specs=pl.BlockSpec((BW, H), lambda i: (i, 0)),
        out_shape=jax.ShapeDtypeStruct((W, H), jnp.float32),
    )(m, pic_tensor)
    return out


# speedup vs baseline: 1.0021x; 1.0021x over previous
import functools

import jax
import jax.numpy as jnp
from jax import lax
from jax.experimental import pallas as pl
from jax.experimental.pallas import tpu as pltpu
from jax.experimental.pallas import tpu_sc as plsc

W = 1280
H = 720
WH = W * H
TWO = 2 * WH
N = 2_000_000
C = 640
R = C // 128
NCHUNK = N // C
NW = 32
SLICE = TWO // 16
ZC = 7200


def _scatter_body(ex, ey, ep, zeros_in, ones_in, masks, shared,
                  xa, ya, pa, xbb, yb, pb, ia, ib, ones_v,
                  lsa, lsb, ssa, ssb):
    c = lax.axis_index("c")
    s = lax.axis_index("s")
    wid = s * 2 + c

    start = wid * NCHUNK // NW
    end = (wid + 1) * NCHUNK // NW
    cnt = end - start

    def fire_loads(j, xr, yr, pr, ls):
        base = j * C
        pltpu.async_copy(ex.at[pl.ds(base, C)], xr, ls)
        pltpu.async_copy(ey.at[pl.ds(base, C)], yr, ls)
        pltpu.async_copy(ep.at[pl.ds(base, C)], pr, ls)

    fire_loads(start, xa, ya, pa, lsa)
    fire_loads(start + 1, xbb, yb, pb, lsb)

    pltpu.sync_copy(ones_in, ones_v)

    pltpu.sync_copy(zeros_in, shared.at[pl.ds(s * SLICE, SLICE)])
    plsc.subcore_barrier()

    def step(j, xr, yr, pr, idxr, ls, ss, first):
        pltpu.make_async_copy(ex.at[pl.ds(0, C)], xr, ls).wait()
        pltpu.make_async_copy(ex.at[pl.ds(0, C)], yr, ls).wait()
        pltpu.make_async_copy(ex.at[pl.ds(0, C)], pr, ls).wait()

        @pl.when(jnp.logical_not(first))
        def _():
            pltpu.make_async_copy(ex.at[pl.ds(0, C)], xr, ss).wait()

        for r in range(R):
            for i8 in range(8):
                off = r * 128 + i8 * 16
                xv = xr[pl.ds(off, 16)]
                yv = yr[pl.ds(off, 16)]
                pv = pr[pl.ds(off, 16)]
                idx = pv * WH + xv * H + yv
                idxr[pl.ds(off, 16)] = idx
        pltpu.async_copy(ones_v, shared.at[idxr], ss)

        @pl.when(j + 2 < end)
        def _():
            fire_loads(j + 2, xr, yr, pr, ls)

    def pair_body(t2, carry):
        j = start + 2 * t2
        step(j, xa, ya, pa, ia, lsa, ssa, t2 == 0)
        step(j + 1, xbb, yb, pb, ib, lsb, ssb, t2 == 0)
        return carry

    lax.fori_loop(0, cnt // 2, pair_body, 0)

    @pl.when(cnt % 2 == 1)
    def _():
        jt = start + 2 * (cnt // 2)
        step(jt, xa, ya, pa, ia, lsa, ssa, jnp.bool_(False))

    pltpu.make_async_copy(ex.at[pl.ds(0, C)], xa, ssa).wait()
    pltpu.make_async_copy(ex.at[pl.ds(0, C)], xbb, ssb).wait()
    plsc.subcore_barrier()

    pltpu.sync_copy(shared.at[pl.ds(s * SLICE, SLICE)],
                    masks.at[pl.ds(c * TWO + s * SLICE, SLICE)])


@functools.partial(
    pl.kernel,
    out_type=jax.ShapeDtypeStruct((2 * TWO,), jnp.int32),
    mesh=plsc.VectorSubcoreMesh(core_axis_name="c", subcore_axis_name="s"),
    scratch_types=[
        pltpu.VMEM_SHARED((TWO,), jnp.int32),
        pltpu.VMEM((C,), jnp.int32),
        pltpu.VMEM((C,), jnp.int32),
        pltpu.VMEM((C,), jnp.int32),
        pltpu.VMEM((C,), jnp.int32),
        pltpu.VMEM((C,), jnp.int32),
        pltpu.VMEM((C,), jnp.int32),
        pltpu.VMEM((C,), jnp.int32),
        pltpu.VMEM((C,), jnp.int32),
        pltpu.VMEM((C,), jnp.int32),
        pltpu.SemaphoreType.DMA,
        pltpu.SemaphoreType.DMA,
        pltpu.SemaphoreType.DMA,
        pltpu.SemaphoreType.DMA,
    ],
)
def _scatter_masks(ex, ey, ep, zeros_in, ones_in, masks, shared,
                   xa, ya, pa, xbb, yb, pb, ia, ib, ones_v,
                   lsa, lsb, ssa, ssb):
    _scatter_body(ex, ey, ep, zeros_in, ones_in, masks, shared,
                  xa, ya, pa, xbb, yb, pb, ia, ib, ones_v,
                  lsa, lsb, ssa, ssb)


def _combine_body(m_ref, pic_ref, o_ref):
    t0 = jnp.maximum(m_ref[0, 0], m_ref[1, 0]).astype(jnp.float32)
    t1 = jnp.maximum(m_ref[0, 1], m_ref[1, 1]).astype(jnp.float32)
    o_ref[...] = pic_ref[...] + 15.0 * t1 - 15.0 * t0


def kernel(events_x, events_y, events_polarity, pic_tensor):
    zeros_in = jnp.zeros((SLICE,), jnp.int32)
    ones_in = jnp.ones((C,), jnp.int32)
    masks = _scatter_masks(events_x, events_y, events_polarity, zeros_in,
                           ones_in)
    m = masks.reshape(2, 2, W, H)
    BW = 128
    out = pl.pallas_call(
        _combine_body,
        grid=(W // BW,),
        in_specs=[
            pl.BlockSpec((2, 2, BW, H), lambda i: (0, 0, i, 0)),
            pl.BlockSpec((BW, H), lambda i: (i, 0)),
        ],
        out_specs=pl.BlockSpec((BW, H), lambda i: (i, 0)),
        out_shape=jax.ShapeDtypeStruct((W, H), jnp.float32),
    )(m, pic_tensor)
    return out


# speedup vs baseline: 1.1968x; 1.1943x over previous
import functools

import jax
import jax.numpy as jnp
from jax import lax
from jax.experimental import pallas as pl
from jax.experimental.pallas import tpu as pltpu
from jax.experimental.pallas import tpu_sc as plsc

W = 1280
H = 720
WH = W * H
TWO = 2 * WH
N = 2_000_000
C = 640
R = C // 128
NCHUNK = N // C
NW = 32
SLICE = TWO // 16
ZC = 7200


def _scatter_body(ex, ey, ep, zeros_in, ones_in, masks, shared,
                  xa, ya, pa, xbb, yb, pb, ia, ib, ones_v,
                  lsa, lsb, ssa, ssb):
    c = lax.axis_index("c")
    s = lax.axis_index("s")
    wid = s * 2 + c

    start = wid * NCHUNK // NW
    end = (wid + 1) * NCHUNK // NW
    cnt = end - start

    def fire_loads(j, xr, yr, pr, ls):
        base = j * C
        pltpu.async_copy(ex.at[pl.ds(base, C)], xr, ls)
        pltpu.async_copy(ey.at[pl.ds(base, C)], yr, ls)
        pltpu.async_copy(ep.at[pl.ds(base, C)], pr, ls)

    fire_loads(start, xa, ya, pa, lsa)
    fire_loads(start + 1, xbb, yb, pb, lsb)

    pltpu.sync_copy(ones_in, ones_v)

    pltpu.sync_copy(zeros_in, shared.at[pl.ds(s * SLICE, SLICE)])
    plsc.subcore_barrier()

    def step(j, xr, yr, pr, idxr, ls, ss, first):
        pltpu.make_async_copy(ex.at[pl.ds(0, C)], xr, ls).wait()
        pltpu.make_async_copy(ex.at[pl.ds(0, C)], yr, ls).wait()
        pltpu.make_async_copy(ex.at[pl.ds(0, C)], pr, ls).wait()

        @pl.when(jnp.logical_not(first))
        def _():
            pltpu.make_async_copy(ex.at[pl.ds(0, C)], xr, ss).wait()

        for r in range(R):
            for i8 in range(8):
                off = r * 128 + i8 * 16
                xv = xr[pl.ds(off, 16)]
                yv = yr[pl.ds(off, 16)]
                pv = pr[pl.ds(off, 16)]
                idx = pv * WH + xv * H + yv
                idxr[pl.ds(off, 16)] = idx
        pltpu.async_copy(ones_v, shared.at[idxr], ss)

        @pl.when(j + 2 < end)
        def _():
            fire_loads(j + 2, xr, yr, pr, ls)

    def pair_body(t2, carry):
        j = start + 2 * t2
        step(j, xa, ya, pa, ia, lsa, ssa, t2 == 0)
        step(j + 1, xbb, yb, pb, ib, lsb, ssb, t2 == 0)
        return carry

    lax.fori_loop(0, cnt // 2, pair_body, 0)

    @pl.when(cnt % 2 == 1)
    def _():
        jt = start + 2 * (cnt // 2)
        step(jt, xa, ya, pa, ia, lsa, ssa, jnp.bool_(False))

    pltpu.make_async_copy(ex.at[pl.ds(0, C)], xa, ssa).wait()
    pltpu.make_async_copy(ex.at[pl.ds(0, C)], xbb, ssb).wait()
    plsc.subcore_barrier()

    pltpu.sync_copy(shared.at[pl.ds(s * SLICE, SLICE)],
                    masks.at[pl.ds(c * TWO + s * SLICE, SLICE)])


@functools.partial(
    pl.kernel,
    out_type=jax.ShapeDtypeStruct((2 * TWO,), jnp.int32),
    mesh=plsc.VectorSubcoreMesh(core_axis_name="c", subcore_axis_name="s"),
    scratch_types=[
        pltpu.VMEM_SHARED((TWO,), jnp.int32),
        pltpu.VMEM((C,), jnp.int32),
        pltpu.VMEM((C,), jnp.int32),
        pltpu.VMEM((C,), jnp.int32),
        pltpu.VMEM((C,), jnp.int32),
        pltpu.VMEM((C,), jnp.int32),
        pltpu.VMEM((C,), jnp.int32),
        pltpu.VMEM((C,), jnp.int32),
        pltpu.VMEM((C,), jnp.int32),
        pltpu.VMEM((C,), jnp.int32),
        pltpu.SemaphoreType.DMA,
        pltpu.SemaphoreType.DMA,
        pltpu.SemaphoreType.DMA,
        pltpu.SemaphoreType.DMA,
    ],
)
def _scatter_masks(ex, ey, ep, zeros_in, ones_in, masks, shared,
                   xa, ya, pa, xbb, yb, pb, ia, ib, ones_v,
                   lsa, lsb, ssa, ssb):
    _scatter_body(ex, ey, ep, zeros_in, ones_in, masks, shared,
                  xa, ya, pa, xbb, yb, pb, ia, ib, ones_v,
                  lsa, lsb, ssa, ssb)


def _combine_body(m00_ref, m01_ref, m10_ref, m11_ref, pic_ref, o_ref):
    t0 = jnp.maximum(m00_ref[...], m10_ref[...]).astype(jnp.float32)
    t1 = jnp.maximum(m01_ref[...], m11_ref[...]).astype(jnp.float32)
    o_ref[...] = pic_ref[...] + 15.0 * t1 - 15.0 * t0


def kernel(events_x, events_y, events_polarity, pic_tensor):
    zeros_in = jnp.zeros((SLICE,), jnp.int32)
    ones_in = jnp.ones((C,), jnp.int32)
    masks = _scatter_masks(events_x, events_y, events_polarity, zeros_in,
                           ones_in)
    pic_flat = pic_tensor.reshape(WH)
    nb = 10
    BLK = WH // nb
    mask_spec = lambda c, p: pl.BlockSpec(
        (BLK,), lambda i, c=c, p=p: (c * 2 * nb + p * nb + i,))
    out = pl.pallas_call(
        _combine_body,
        grid=(nb,),
        in_specs=[
            mask_spec(0, 0),
            mask_spec(0, 1),
            mask_spec(1, 0),
            mask_spec(1, 1),
            pl.BlockSpec((BLK,), lambda i: (i,)),
        ],
        out_specs=pl.BlockSpec((BLK,), lambda i: (i,)),
        out_shape=jax.ShapeDtypeStruct((WH,), jnp.float32),
    )(masks, masks, masks, masks, pic_flat)
    return out.reshape(W, H)


# speedup vs baseline: 1.2143x; 1.0147x over previous
import functools

import numpy as np

import jax
import jax.numpy as jnp
from jax import lax
from jax.experimental import pallas as pl
from jax.experimental.pallas import tpu as pltpu
from jax.experimental.pallas import tpu_sc as plsc

W = 1280
H = 720
WH = W * H
TWO = 2 * WH
N = 2_000_000
C = 640
R = C // 128
NCHUNK = N // C
NW = 32
SLICE = TWO // 16
ZC = 7200


def _scatter_body(ex, ey, ep, zeros_in, ones_in, masks, shared,
                  xa, ya, pa, xbb, yb, pb, ia, ib, ones_v,
                  lsa, lsb, ssa, ssb):
    c = lax.axis_index("c")
    s = lax.axis_index("s")
    wid = s * 2 + c

    start = wid * NCHUNK // NW
    end = (wid + 1) * NCHUNK // NW
    cnt = end - start

    def fire_loads(j, xr, yr, pr, ls):
        base = j * C
        pltpu.async_copy(ex.at[pl.ds(base, C)], xr, ls)
        pltpu.async_copy(ey.at[pl.ds(base, C)], yr, ls)
        pltpu.async_copy(ep.at[pl.ds(base, C)], pr, ls)

    fire_loads(start, xa, ya, pa, lsa)
    fire_loads(start + 1, xbb, yb, pb, lsb)

    pltpu.sync_copy(ones_in, ones_v)

    pltpu.sync_copy(zeros_in, shared.at[pl.ds(s * SLICE, SLICE)])
    plsc.subcore_barrier()

    def step(j, xr, yr, pr, idxr, ls, ss, first):
        pltpu.make_async_copy(ex.at[pl.ds(0, C)], xr, ls).wait()
        pltpu.make_async_copy(ex.at[pl.ds(0, C)], yr, ls).wait()
        pltpu.make_async_copy(ex.at[pl.ds(0, C)], pr, ls).wait()

        @pl.when(jnp.logical_not(first))
        def _():
            pltpu.make_async_copy(ex.at[pl.ds(0, C)], xr, ss).wait()

        for r in range(R):
            for i8 in range(8):
                off = r * 128 + i8 * 16
                xv = xr[pl.ds(off, 16)]
                yv = yr[pl.ds(off, 16)]
                pv = pr[pl.ds(off, 16)]
                idx = pv * WH + xv * H + yv
                idxr[pl.ds(off, 16)] = idx
        pltpu.async_copy(ones_v, shared.at[idxr], ss)

        @pl.when(j + 2 < end)
        def _():
            fire_loads(j + 2, xr, yr, pr, ls)

    def pair_body(t2, carry):
        j = start + 2 * t2
        step(j, xa, ya, pa, ia, lsa, ssa, t2 == 0)
        step(j + 1, xbb, yb, pb, ib, lsb, ssb, t2 == 0)
        return carry

    lax.fori_loop(0, cnt // 2, pair_body, 0)

    @pl.when(cnt % 2 == 1)
    def _():
        jt = start + 2 * (cnt // 2)
        step(jt, xa, ya, pa, ia, lsa, ssa, jnp.bool_(False))

    pltpu.make_async_copy(ex.at[pl.ds(0, C)], xa, ssa).wait()
    pltpu.make_async_copy(ex.at[pl.ds(0, C)], xbb, ssb).wait()
    plsc.subcore_barrier()

    pltpu.sync_copy(shared.at[pl.ds(s * SLICE, SLICE)],
                    masks.at[pl.ds(c * TWO + s * SLICE, SLICE)])


@functools.partial(
    pl.kernel,
    out_type=jax.ShapeDtypeStruct((2 * TWO,), jnp.int32),
    mesh=plsc.VectorSubcoreMesh(core_axis_name="c", subcore_axis_name="s"),
    scratch_types=[
        pltpu.VMEM_SHARED((TWO,), jnp.int32),
        pltpu.VMEM((C,), jnp.int32),
        pltpu.VMEM((C,), jnp.int32),
        pltpu.VMEM((C,), jnp.int32),
        pltpu.VMEM((C,), jnp.int32),
        pltpu.VMEM((C,), jnp.int32),
        pltpu.VMEM((C,), jnp.int32),
        pltpu.VMEM((C,), jnp.int32),
        pltpu.VMEM((C,), jnp.int32),
        pltpu.VMEM((C,), jnp.int32),
        pltpu.SemaphoreType.DMA,
        pltpu.SemaphoreType.DMA,
        pltpu.SemaphoreType.DMA,
        pltpu.SemaphoreType.DMA,
    ],
)
def _scatter_masks(ex, ey, ep, zeros_in, ones_in, masks, shared,
                   xa, ya, pa, xbb, yb, pb, ia, ib, ones_v,
                   lsa, lsb, ssa, ssb):
    _scatter_body(ex, ey, ep, zeros_in, ones_in, masks, shared,
                  xa, ya, pa, xbb, yb, pb, ia, ib, ones_v,
                  lsa, lsb, ssa, ssb)


_ZEROS = np.zeros((SLICE,), np.int32)
_ONES = np.ones((C,), np.int32)


def _combine_body(m00_ref, m01_ref, m10_ref, m11_ref, pic_ref, o_ref):
    t0 = jnp.maximum(m00_ref[...], m10_ref[...]).astype(jnp.float32)
    t1 = jnp.maximum(m01_ref[...], m11_ref[...]).astype(jnp.float32)
    o_ref[...] = pic_ref[...] + 15.0 * t1 - 15.0 * t0


def kernel(events_x, events_y, events_polarity, pic_tensor):
    zeros_in = jnp.asarray(_ZEROS)
    ones_in = jnp.asarray(_ONES)
    masks = _scatter_masks(events_x, events_y, events_polarity, zeros_in,
                           ones_in)
    pic_flat = pic_tensor.reshape(WH)
    nb = 5
    BLK = WH // nb
    mask_spec = lambda c, p: pl.BlockSpec(
        (BLK,), lambda i, c=c, p=p: (c * 2 * nb + p * nb + i,))
    out = pl.pallas_call(
        _combine_body,
        grid=(nb,),
        in_specs=[
            mask_spec(0, 0),
            mask_spec(0, 1),
            mask_spec(1, 0),
            mask_spec(1, 1),
            pl.BlockSpec((BLK,), lambda i: (i,)),
        ],
        out_specs=pl.BlockSpec((BLK,), lambda i: (i,)),
        out_shape=jax.ShapeDtypeStruct((WH,), jnp.float32),
    )(masks, masks, masks, masks, pic_flat)
    return out.reshape(W, H)


# speedup vs baseline: 1.3300x; 1.0953x over previous
import functools

import numpy as np

import jax
import jax.numpy as jnp
from jax import lax
from jax.experimental import pallas as pl
from jax.experimental.pallas import tpu as pltpu
from jax.experimental.pallas import tpu_sc as plsc

W = 1280
H = 720
WH = W * H
TWO = 2 * WH
N = 2_000_000
C = 640
R = C // 128
NCHUNK = N // C
NW = 32
SLICE = TWO // 16
ZC = 7200


def _scatter_body(ex, ey, ep, zeros_in, ones_in, masks, shared,
                  xa, ya, pa, xbb, yb, pb, ia, ib, ones_v, zbuf,
                  lsa, lsb, ssa, ssb):
    c = lax.axis_index("c")
    s = lax.axis_index("s")
    wid = s * 2 + c

    start = wid * NCHUNK // NW
    end = (wid + 1) * NCHUNK // NW
    cnt = end - start

    def fire_loads(j, xr, yr, pr, ls):
        base = j * C
        pltpu.async_copy(ex.at[pl.ds(base, C)], xr, ls)
        pltpu.async_copy(ey.at[pl.ds(base, C)], yr, ls)
        pltpu.async_copy(ep.at[pl.ds(base, C)], pr, ls)

    fire_loads(start, xa, ya, pa, lsa)
    fire_loads(start + 1, xbb, yb, pb, lsb)

    pltpu.sync_copy(ones_in, ones_v)

    pltpu.sync_copy(zeros_in, zbuf)

    def zcopy(k, carry):
        pltpu.async_copy(zbuf, shared.at[pl.ds(s * SLICE + k * ZC, ZC)], ssa)
        return carry

    lax.fori_loop(0, SLICE // ZC, zcopy, 0)
    pltpu.make_async_copy(ex.at[pl.ds(0, C)],
                          shared.at[pl.ds(s * SLICE, SLICE)], ssa).wait()
    plsc.subcore_barrier()

    def step(j, xr, yr, pr, idxr, ls, ss, first):
        pltpu.make_async_copy(ex.at[pl.ds(0, C)], xr, ls).wait()
        pltpu.make_async_copy(ex.at[pl.ds(0, C)], yr, ls).wait()
        pltpu.make_async_copy(ex.at[pl.ds(0, C)], pr, ls).wait()

        @pl.when(jnp.logical_not(first))
        def _():
            pltpu.make_async_copy(ex.at[pl.ds(0, C)], xr, ss).wait()

        for r in range(R):
            for i8 in range(8):
                off = r * 128 + i8 * 16
                xv = xr[pl.ds(off, 16)]
                yv = yr[pl.ds(off, 16)]
                pv = pr[pl.ds(off, 16)]
                idx = pv * WH + xv * H + yv
                idxr[pl.ds(off, 16)] = idx
        pltpu.async_copy(ones_v, shared.at[idxr], ss)

        @pl.when(j + 2 < end)
        def _():
            fire_loads(j + 2, xr, yr, pr, ls)

    def pair_body(t2, carry):
        j = start + 2 * t2
        step(j, xa, ya, pa, ia, lsa, ssa, t2 == 0)
        step(j + 1, xbb, yb, pb, ib, lsb, ssb, t2 == 0)
        return carry

    lax.fori_loop(0, cnt // 2, pair_body, 0)

    @pl.when(cnt % 2 == 1)
    def _():
        jt = start + 2 * (cnt // 2)
        step(jt, xa, ya, pa, ia, lsa, ssa, jnp.bool_(False))

    pltpu.make_async_copy(ex.at[pl.ds(0, C)], xa, ssa).wait()
    pltpu.make_async_copy(ex.at[pl.ds(0, C)], xbb, ssb).wait()
    plsc.subcore_barrier()

    pltpu.sync_copy(shared.at[pl.ds(s * SLICE, SLICE)],
                    masks.at[pl.ds(c * TWO + s * SLICE, SLICE)])


@functools.partial(
    pl.kernel,
    out_type=jax.ShapeDtypeStruct((2 * TWO,), jnp.int32),
    mesh=plsc.VectorSubcoreMesh(core_axis_name="c", subcore_axis_name="s"),
    scratch_types=[
        pltpu.VMEM_SHARED((TWO,), jnp.int32),
        pltpu.VMEM((C,), jnp.int32),
        pltpu.VMEM((C,), jnp.int32),
        pltpu.VMEM((C,), jnp.int32),
        pltpu.VMEM((C,), jnp.int32),
        pltpu.VMEM((C,), jnp.int32),
        pltpu.VMEM((C,), jnp.int32),
        pltpu.VMEM((C,), jnp.int32),
        pltpu.VMEM((C,), jnp.int32),
        pltpu.VMEM((C,), jnp.int32),
        pltpu.VMEM((ZC,), jnp.int32),
        pltpu.SemaphoreType.DMA,
        pltpu.SemaphoreType.DMA,
        pltpu.SemaphoreType.DMA,
        pltpu.SemaphoreType.DMA,
    ],
)
def _scatter_masks(ex, ey, ep, zeros_in, ones_in, masks, shared,
                   xa, ya, pa, xbb, yb, pb, ia, ib, ones_v, zbuf,
                   lsa, lsb, ssa, ssb):
    _scatter_body(ex, ey, ep, zeros_in, ones_in, masks, shared,
                  xa, ya, pa, xbb, yb, pb, ia, ib, ones_v, zbuf,
                  lsa, lsb, ssa, ssb)


_ZEROS = np.zeros((ZC,), np.int32)
_ONES = np.ones((C,), np.int32)


def _combine_body(m00_ref, m01_ref, m10_ref, m11_ref, pic_ref, o_ref):
    t0 = jnp.maximum(m00_ref[...], m10_ref[...]).astype(jnp.float32)
    t1 = jnp.maximum(m01_ref[...], m11_ref[...]).astype(jnp.float32)
    o_ref[...] = pic_ref[...] + 15.0 * t1 - 15.0 * t0


def kernel(events_x, events_y, events_polarity, pic_tensor):
    zeros_in = jnp.asarray(_ZEROS)
    ones_in = jnp.asarray(_ONES)
    masks = _scatter_masks(events_x, events_y, events_polarity, zeros_in,
                           ones_in)
    pic_flat = pic_tensor.reshape(WH)
    nb = 5
    BLK = WH // nb
    mask_spec = lambda c, p: pl.BlockSpec(
        (BLK,), lambda i, c=c, p=p: (c * 2 * nb + p * nb + i,))
    out = pl.pallas_call(
        _combine_body,
        grid=(nb,),
        in_specs=[
            mask_spec(0, 0),
            mask_spec(0, 1),
            mask_spec(1, 0),
            mask_spec(1, 1),
            pl.BlockSpec((BLK,), lambda i: (i,)),
        ],
        out_specs=pl.BlockSpec((BLK,), lambda i: (i,)),
        out_shape=jax.ShapeDtypeStruct((WH,), jnp.float32),
        input_output_aliases={4: 0},
    )(masks, masks, masks, masks, pic_flat)
    return out.reshape(W, H)
